# baseline (speedup 1.0000x reference)
import functools

import jax
import jax.numpy as jnp
from jax import lax
from jax.experimental import pallas as pl
from jax.experimental.pallas import tpu as pltpu
from jax.experimental.pallas import tpu_sc as plsc

NC = 2
NS = 16
L = 16
NW = NC * NS

N_FEATS = 100000
NNZ = 3200000
N_OBS = 3200000

NBINS = 102400
E = 2 * NNZ
NE = E // NW
CH = 8000
NCH = NE // CH
US = 10
NO = N_OBS // NW
CG = 4000
NCG = NO // CG
UG = 10

ROWS = NBINS // 128
GRID0 = ROWS // 8

_mesh = plsc.VectorSubcoreMesh(core_axis_name="c", subcore_axis_name="s")
_params = pltpu.CompilerParams(needs_layout_passes=False,
                               use_tc_tiling_on_sc=False)


@functools.partial(
    pl.kernel, mesh=_mesh, compiler_params=_params,
    out_type=jax.ShapeDtypeStruct((NW, NBINS), jnp.float32),
    scratch_types=[
        pltpu.VMEM((NBINS,), jnp.float32),
        pltpu.VMEM((CH,), jnp.int32),
        pltpu.VMEM((CH,), jnp.int32),
        pltpu.SemaphoreType.DMA,
        pltpu.SemaphoreType.DMA,
    ],
)
def _k1_histogram(idx_hbm, out_hbm, hist, idxa, idxb, sem0, sem1):
    c = lax.axis_index("c")
    s = lax.axis_index("s")
    wid = s * NC + c

    @plsc.parallel_loop(0, NBINS // L, 1, unroll=16)
    def _(i):
        hist[pl.ds(i * L, L)] = jnp.zeros((L,), jnp.float32)

    bufs = (idxa, idxb)
    sems = (sem0, sem1)
    tile_base = wid * NE
    cps = [None, None]
    cps[0] = pltpu.async_copy(idx_hbm.at[pl.ds(tile_base, CH)], idxa, sem0)
    for k in range(NCH):
        cur = k % 2
        if k + 1 < NCH:
            nxt = (k + 1) % 2
            cps[nxt] = pltpu.async_copy(
                idx_hbm.at[pl.ds(tile_base + (k + 1) * CH, CH)],
                bufs[nxt], sems[nxt])
        cps[cur].wait()
        buf = bufs[cur]

        @plsc.parallel_loop(0, CH // L, 1, unroll=US)
        def _(i):
            iv = buf[pl.ds(i * L, L)]
            plsc.addupdate_scatter(hist, [iv], jnp.ones((L,), jnp.float32))

    pltpu.sync_copy(hist, out_hbm.at[wid])


@functools.partial(
    pl.pallas_call,
    grid=(2 * GRID0,),
    in_specs=[
        pl.BlockSpec((NW, 8, 128), lambda i: (0, jnp.minimum(i, GRID0 - 1), 0)),
        pl.BlockSpec((8, 128), lambda i: (jnp.maximum(i - GRID0, 0), 0)),
    ],
    out_specs=pl.BlockSpec((8, 128), lambda i: (jnp.maximum(i - GRID0, 0), 0)),
    out_shape=jax.ShapeDtypeStruct((ROWS, 128), jnp.float32),
    scratch_shapes=[
        pltpu.VMEM((ROWS, 128), jnp.float32),
        pltpu.SMEM((1,), jnp.float32),
    ],
)
def _kt_reduce_t(parts_ref, lam_ref, t_ref, deg_s, max_s):
    i = pl.program_id(0)

    @pl.when(i < GRID0)
    def _():
        d = jnp.sum(parts_ref[...], axis=0)
        deg_s[pl.ds(i * 8, 8), :] = d
        bm = jnp.max(d)

        @pl.when(i == 0)
        def _():
            max_s[0] = bm

        max_s[0] = jnp.maximum(max_s[0], bm)

    @pl.when(i >= GRID0)
    def _():
        j = i - GRID0
        d = deg_s[pl.ds(j * 8, 8), :]
        inv = 1.0 / max_s[0]
        t_ref[...] = 10.0 / (jnp.abs(lam_ref[...] * d * inv) + 1.0) + 1e-05


@functools.partial(
    pl.kernel, mesh=_mesh, compiler_params=_params,
    out_type=jax.ShapeDtypeStruct((N_OBS,), jnp.float32),
    scratch_types=[
        pltpu.VMEM((NBINS,), jnp.float32),
        pltpu.VMEM((CG,), jnp.int32),
        pltpu.VMEM((CG,), jnp.int32),
        pltpu.VMEM((CG,), jnp.float32),
        pltpu.VMEM((CG,), jnp.float32),
        pltpu.SemaphoreType.DMA,
        pltpu.SemaphoreType.DMA,
        pltpu.SemaphoreType.DMA,
        pltpu.SemaphoreType.DMA,
    ],
)
def _k2_gather(t_hbm, wh_hbm, out_hbm, ttab, wha, whb, oba, obb,
               sem0, sem1, sem2, sem3):
    c = lax.axis_index("c")
    s = lax.axis_index("s")
    wid = s * NC + c

    pltpu.sync_copy(t_hbm, ttab)

    whs = (wha, whb)
    outs = (oba, obb)
    isems = (sem0, sem1)
    osems = (sem2, sem3)
    obase = wid * NO
    cin = [None, None]
    cout = [None, None]
    cin[0] = pltpu.async_copy(wh_hbm.at[pl.ds(obase, CG)], wha, sem0)
    for k in range(NCG):
        cur = k % 2
        if k + 1 < NCG:
            nxt = (k + 1) % 2
            cin[nxt] = pltpu.async_copy(
                wh_hbm.at[pl.ds(obase + (k + 1) * CG, CG)],
                whs[nxt], isems[nxt])
        cin[cur].wait()
        if k >= 2:
            cout[cur].wait()
        wh = whs[cur]
        ob = outs[cur]

        @plsc.parallel_loop(0, CG // L, 1, unroll=UG)
        def _(i):
            q = i * L
            iv = wh[pl.ds(q, L)]
            ob[pl.ds(q, L)] = plsc.load_gather(ttab, [iv])

        cout[cur] = pltpu.async_copy(
            ob, out_hbm.at[pl.ds(obase + k * CG, CG)], osems[cur])
    cout[0].wait()
    cout[1].wait()


def kernel(lam, idx, wh_o):
    idx_perm = jnp.reshape(
        jnp.transpose(jnp.reshape(idx, (2, NNZ // 128, 128)), (1, 0, 2)),
        (E,))
    parts = _k1_histogram(idx_perm)
    lam_pad = jnp.reshape(jnp.pad(lam, (0, NBINS - N_FEATS)), (ROWS, 128))
    t3 = _kt_reduce_t(jnp.reshape(parts, (NW, ROWS, 128)), lam_pad)
    return _k2_gather(jnp.reshape(t3, (NBINS,)), wh_o)


# speedup vs baseline: 1.3567x; 1.3567x over previous
import functools

import jax
import jax.numpy as jnp
from jax import lax
from jax.experimental import pallas as pl
from jax.experimental.pallas import tpu as pltpu
from jax.experimental.pallas import tpu_sc as plsc

NC = 2
NS = 16
L = 16
NW = NC * NS

N_FEATS = 100000
NNZ = 3200000
N_OBS = 3200000

NBINS = 102400
E = 2 * NNZ
NE = E // NW
CH = 8000
NCH = NE // CH
US = 10
NO = N_OBS // NW
CG = 4000
NCG = NO // CG
UG = 10

ROWS = NBINS // 128
GRID0 = ROWS // 8

_mesh = plsc.VectorSubcoreMesh(core_axis_name="c", subcore_axis_name="s")
_params = pltpu.CompilerParams(needs_layout_passes=False,
                               use_tc_tiling_on_sc=False)


@functools.partial(
    pl.kernel, mesh=_mesh, compiler_params=_params,
    out_type=jax.ShapeDtypeStruct((NW, NBINS), jnp.float32),
    scratch_types=[
        pltpu.VMEM((NBINS,), jnp.float32),
        pltpu.VMEM((CH,), jnp.int32),
        pltpu.VMEM((CH,), jnp.int32),
        pltpu.SemaphoreType.DMA,
        pltpu.SemaphoreType.DMA,
    ],
)
def _k1_histogram(idx_hbm, out_hbm, hist, idxa, idxb, sem0, sem1):
    c = lax.axis_index("c")
    s = lax.axis_index("s")
    wid = s * NC + c

    @plsc.parallel_loop(0, NBINS // L, 1, unroll=16)
    def _(i):
        hist[pl.ds(i * L, L)] = jnp.zeros((L,), jnp.float32)

    bufs = (idxa, idxb)
    sems = (sem0, sem1)
    tile_base = wid * NE
    cps = [None, None]
    cps[0] = pltpu.async_copy(idx_hbm.at[pl.ds(tile_base, CH)], idxa, sem0)
    for k in range(NCH):
        cur = k % 2
        if k + 1 < NCH:
            nxt = (k + 1) % 2
            cps[nxt] = pltpu.async_copy(
                idx_hbm.at[pl.ds(tile_base + (k + 1) * CH, CH)],
                bufs[nxt], sems[nxt])
        cps[cur].wait()
        buf = bufs[cur]

        @plsc.parallel_loop(0, CH // L, 1, unroll=US)
        def _(i):
            iv = buf[pl.ds(i * L, L)]
            plsc.addupdate_scatter(hist, [iv], jnp.ones((L,), jnp.float32))

    pltpu.sync_copy(hist, out_hbm.at[wid])


@functools.partial(
    pl.pallas_call,
    grid=(GRID0,),
    in_specs=[
        pl.BlockSpec((NW, 8, 128), lambda i: (0, i, 0)),
    ],
    out_specs=[
        pl.BlockSpec((8, 128), lambda i: (i, 0)),
        pl.BlockSpec((8, 128), lambda i: (0, 0)),
    ],
    out_shape=[
        jax.ShapeDtypeStruct((ROWS, 128), jnp.float32),
        jax.ShapeDtypeStruct((8, 128), jnp.float32),
    ],
    scratch_shapes=[
        pltpu.SMEM((1,), jnp.float32),
    ],
)
def _kt_reduce(parts_ref, deg_ref, max_ref, max_s):
    i = pl.program_id(0)
    d = jnp.sum(parts_ref[...], axis=0)
    deg_ref[...] = d
    bm = jnp.max(d)

    @pl.when(i == 0)
    def _():
        max_s[0] = bm

    max_s[0] = jnp.maximum(max_s[0], bm)
    max_ref[...] = jnp.full((8, 128), max_s[0], jnp.float32)


SLICE = NBINS // NS
LAM_TAIL = N_FEATS - (NS - 1) * SLICE


@functools.partial(
    pl.kernel, mesh=_mesh, compiler_params=_params,
    out_type=jax.ShapeDtypeStruct((N_OBS,), jnp.float32),
    scratch_types=[
        pltpu.VMEM((NBINS,), jnp.float32),
        pltpu.VMEM((CG,), jnp.int32),
        pltpu.VMEM((CG,), jnp.int32),
        pltpu.VMEM((CG,), jnp.float32),
        pltpu.VMEM((CG,), jnp.float32),
        pltpu.VMEM((L,), jnp.float32),
        pltpu.VMEM_SHARED((NBINS,), jnp.float32),
        pltpu.SemaphoreType.DMA,
        pltpu.SemaphoreType.DMA,
        pltpu.SemaphoreType.DMA,
        pltpu.SemaphoreType.DMA,
    ],
)
def _k2_normalize_gather(deg_hbm, max_hbm, lam_hbm, wh_hbm, out_hbm,
                         ttab, wha, whb, oba, obb, mbuf, st,
                         sem0, sem1, sem2, sem3):
    c = lax.axis_index("c")
    s = lax.axis_index("s")
    wid = s * NC + c
    off = s * SLICE

    pltpu.sync_copy(deg_hbm.at[pl.ds(off, SLICE)], ttab.at[pl.ds(0, SLICE)])
    pltpu.sync_copy(max_hbm.at[pl.ds(0, L)], mbuf)

    @pl.when(s < NS - 1)
    def _():
        pltpu.sync_copy(lam_hbm.at[pl.ds(off, SLICE)],
                        ttab.at[pl.ds(2 * SLICE, SLICE)])

    @pl.when(s == NS - 1)
    def _():
        pltpu.sync_copy(lam_hbm.at[pl.ds(off, LAM_TAIL)],
                        ttab.at[pl.ds(2 * SLICE, LAM_TAIL)])

    inv = 1.0 / mbuf[pl.ds(0, L)]

    @plsc.parallel_loop(0, SLICE // L, 1, unroll=8)
    def _(j):
        q = j * L
        d = ttab[pl.ds(q, L)]
        lm = ttab[pl.ds(2 * SLICE + q, L)]
        ttab[pl.ds(SLICE + q, L)] = (
            10.0 / (jnp.abs(lm * d * inv) + 1.0) + 1e-05)

    pltpu.sync_copy(ttab.at[pl.ds(SLICE, SLICE)], st.at[pl.ds(off, SLICE)])
    plsc.subcore_barrier()
    pltpu.sync_copy(st, ttab)

    whs = (wha, whb)
    outs = (oba, obb)
    isems = (sem0, sem1)
    osems = (sem2, sem3)
    obase = wid * NO
    cin = [None, None]
    cout = [None, None]
    cin[0] = pltpu.async_copy(wh_hbm.at[pl.ds(obase, CG)], wha, sem0)
    for k in range(NCG):
        cur = k % 2
        if k + 1 < NCG:
            nxt = (k + 1) % 2
            cin[nxt] = pltpu.async_copy(
                wh_hbm.at[pl.ds(obase + (k + 1) * CG, CG)],
                whs[nxt], isems[nxt])
        cin[cur].wait()
        if k >= 2:
            cout[cur].wait()
        wh = whs[cur]
        ob = outs[cur]

        @plsc.parallel_loop(0, CG // L, 1, unroll=UG)
        def _(i):
            q = i * L
            iv = wh[pl.ds(q, L)]
            ob[pl.ds(q, L)] = plsc.load_gather(ttab, [iv])

        cout[cur] = pltpu.async_copy(
            ob, out_hbm.at[pl.ds(obase + k * CG, CG)], osems[cur])
    cout[0].wait()
    cout[1].wait()


def kernel(lam, idx, wh_o):
    idx_perm = jnp.reshape(
        jnp.transpose(jnp.reshape(idx, (2, NNZ // 128, 128)), (1, 0, 2)),
        (E,))
    parts = _k1_histogram(idx_perm)
    deg3, max3 = _kt_reduce(jnp.reshape(parts, (NW, ROWS, 128)))
    return _k2_normalize_gather(jnp.reshape(deg3, (NBINS,)),
                                jnp.reshape(max3, (8 * 128,)),
                                lam, wh_o)


# speedup vs baseline: 1.6355x; 1.2055x over previous
import functools

import jax
import jax.numpy as jnp
from jax import lax
from jax.experimental import pallas as pl
from jax.experimental.pallas import tpu as pltpu
from jax.experimental.pallas import tpu_sc as plsc

NC = 2
NS = 16
L = 16
NW = NC * NS

N_FEATS = 100000
NNZ = 3200000
N_OBS = 3200000

NBINS = 102400
SLICE = NBINS // NS
LAM_TAIL = N_FEATS - (NS - 1) * SLICE
E = 2 * NNZ
NE = E // NW
CH = 8000
NCH = NE // CH
US = 20
NO = N_OBS // NW
CG = 4000
NCG = NO // CG
UG = 10
UA = 8

_mesh = plsc.VectorSubcoreMesh(core_axis_name="c", subcore_axis_name="s")
_params = pltpu.CompilerParams(needs_layout_passes=False,
                               use_tc_tiling_on_sc=False)


@functools.partial(
    pl.kernel, mesh=_mesh, compiler_params=_params,
    out_type=jax.ShapeDtypeStruct((NW, NBINS), jnp.float32),
    scratch_types=[
        pltpu.VMEM((NBINS,), jnp.float32),
        pltpu.VMEM((CH,), jnp.int32),
        pltpu.VMEM((CH,), jnp.int32),
        pltpu.SemaphoreType.DMA,
        pltpu.SemaphoreType.DMA,
    ],
)
def _k1_histogram(idx_hbm, out_hbm, hist, idxa, idxb, sem0, sem1):
    c = lax.axis_index("c")
    s = lax.axis_index("s")
    wid = s * NC + c

    @plsc.parallel_loop(0, NBINS // L, 1, unroll=16)
    def _(i):
        hist[pl.ds(i * L, L)] = jnp.zeros((L,), jnp.float32)

    bufs = (idxa, idxb)
    sems = (sem0, sem1)
    tile_base = wid * NE
    cps = [None, None]
    cps[0] = pltpu.async_copy(idx_hbm.at[pl.ds(tile_base, CH)], idxa, sem0)
    for k in range(NCH):
        cur = k % 2
        if k + 1 < NCH:
            nxt = (k + 1) % 2
            cps[nxt] = pltpu.async_copy(
                idx_hbm.at[pl.ds(tile_base + (k + 1) * CH, CH)],
                bufs[nxt], sems[nxt])
        cps[cur].wait()
        buf = bufs[cur]

        @plsc.parallel_loop(0, CH // L, 1, unroll=US)
        def _(i):
            iv = buf[pl.ds(i * L, L)]
            plsc.addupdate_scatter(hist, [iv], jnp.ones((L,), jnp.float32))

    pltpu.sync_copy(hist, out_hbm.at[wid])


@functools.partial(
    pl.kernel, mesh=_mesh, compiler_params=_params,
    out_type=jax.ShapeDtypeStruct((N_OBS,), jnp.float32),
    scratch_types=[
        pltpu.VMEM((NBINS,), jnp.float32),
        pltpu.VMEM((CG,), jnp.int32),
        pltpu.VMEM((CG,), jnp.int32),
        pltpu.VMEM((SLICE,), jnp.float32),
        pltpu.VMEM((SLICE,), jnp.float32),
        pltpu.VMEM((NS * L,), jnp.float32),
        pltpu.VMEM_SHARED((NBINS,), jnp.float32),
        pltpu.VMEM_SHARED((NS * L,), jnp.float32),
        pltpu.SemaphoreType.DMA,
        pltpu.SemaphoreType.DMA,
        pltpu.SemaphoreType.DMA,
        pltpu.SemaphoreType.DMA,
    ],
)
def _k2_normalize_gather(parts_hbm, lam_hbm, wh_hbm, out_hbm,
                         ttab, wha, whb, tbuf, tbuf2, mbuf, st, smax,
                         sem0, sem1, sem2, sem3):
    c = lax.axis_index("c")
    s = lax.axis_index("s")
    wid = s * NC + c
    off = s * SLICE

    pltpu.sync_copy(parts_hbm.at[0, pl.ds(off, SLICE)], ttab.at[pl.ds(0, SLICE)])
    cp1 = pltpu.async_copy(parts_hbm.at[1, pl.ds(off, SLICE)], tbuf, sem0)
    cp2 = pltpu.async_copy(parts_hbm.at[2, pl.ds(off, SLICE)], tbuf2, sem1)
    for p in range(1, NW):
        use_a = (p % 2) == 1
        (cp1 if use_a else cp2).wait()
        src = tbuf if use_a else tbuf2

        @plsc.parallel_loop(0, SLICE // L, 1, unroll=UA)
        def _(j):
            q = j * L
            ttab[pl.ds(q, L)] = ttab[pl.ds(q, L)] + src[pl.ds(q, L)]

        if p + 2 < NW:
            if use_a:
                cp1 = pltpu.async_copy(parts_hbm.at[p + 2, pl.ds(off, SLICE)],
                                       tbuf, sem0)
            else:
                cp2 = pltpu.async_copy(parts_hbm.at[p + 2, pl.ds(off, SLICE)],
                                       tbuf2, sem1)

    @pl.when(s < NS - 1)
    def _():
        pltpu.sync_copy(lam_hbm.at[pl.ds(off, SLICE)],
                        ttab.at[pl.ds(2 * SLICE, SLICE)])

    @pl.when(s == NS - 1)
    def _():
        pltpu.sync_copy(lam_hbm.at[pl.ds(off, LAM_TAIL)],
                        ttab.at[pl.ds(2 * SLICE, LAM_TAIL)])

    def max_body(j, m):
        return jnp.maximum(m, ttab[pl.ds(j * L, L)])
    mv = plsc.parallel_loop(0, SLICE // L, 1, unroll=UA,
                            carry=jnp.zeros((L,), jnp.float32))(max_body)
    lmax = lax.reduce_max_p.bind(mv, axes=(0,))
    mbuf[pl.ds(0, L)] = jnp.full((L,), lmax)
    pltpu.sync_copy(mbuf.at[pl.ds(0, L)], smax.at[pl.ds(s * L, L)])
    plsc.subcore_barrier()
    pltpu.sync_copy(smax, mbuf)
    gv = mbuf[pl.ds(0, L)]
    for p in range(1, NS):
        gv = jnp.maximum(gv, mbuf[pl.ds(p * L, L)])
    inv = 1.0 / jnp.full((L,), lax.reduce_max_p.bind(gv, axes=(0,)))

    @plsc.parallel_loop(0, SLICE // L, 1, unroll=UA)
    def _(j):
        q = j * L
        d = ttab[pl.ds(q, L)]
        lm = ttab[pl.ds(2 * SLICE + q, L)]
        ttab[pl.ds(SLICE + q, L)] = (
            10.0 / (jnp.abs(lm * d * inv) + 1.0) + 1e-05)

    pltpu.sync_copy(ttab.at[pl.ds(SLICE, SLICE)], st.at[pl.ds(off, SLICE)])
    plsc.subcore_barrier()
    pltpu.sync_copy(st, ttab)

    whs = (wha, whb)
    outs = (tbuf, tbuf2)
    isems = (sem0, sem1)
    osems = (sem2, sem3)
    obase = wid * NO
    cin = [None, None]
    cout = [None, None]
    cin[0] = pltpu.async_copy(wh_hbm.at[pl.ds(obase, CG)], wha, sem0)
    for k in range(NCG):
        cur = k % 2
        if k + 1 < NCG:
            nxt = (k + 1) % 2
            cin[nxt] = pltpu.async_copy(
                wh_hbm.at[pl.ds(obase + (k + 1) * CG, CG)],
                whs[nxt], isems[nxt])
        cin[cur].wait()
        if k >= 2:
            cout[cur].wait()
        wh = whs[cur]
        ob = outs[cur]

        @plsc.parallel_loop(0, CG // L, 1, unroll=UG)
        def _(i):
            q = i * L
            iv = wh[pl.ds(q, L)]
            ob[pl.ds(q, L)] = plsc.load_gather(ttab, [iv])

        cout[cur] = pltpu.async_copy(
            ob.at[pl.ds(0, CG)], out_hbm.at[pl.ds(obase + k * CG, CG)],
            osems[cur])
    cout[0].wait()
    cout[1].wait()


def kernel(lam, idx, wh_o):
    idx_perm = jnp.reshape(
        jnp.transpose(jnp.reshape(idx, (2, NNZ // 128, 128)), (1, 0, 2)),
        (E,))
    parts = _k1_histogram(idx_perm)
    return _k2_normalize_gather(parts, lam, wh_o)


# speedup vs baseline: 1.6686x; 1.0203x over previous
import functools

import jax
import jax.numpy as jnp
from jax import lax
from jax.experimental import pallas as pl
from jax.experimental.pallas import tpu as pltpu
from jax.experimental.pallas import tpu_sc as plsc

NC = 2
NS = 16
L = 16
NW = NC * NS

N_FEATS = 100000
NNZ = 3200000
N_OBS = 3200000

NBINS = 102400
SLICE = NBINS // NS
LAM_TAIL = N_FEATS - (NS - 1) * SLICE
E = 2 * NNZ
NE = E // NW
CH = 8000
NCH = NE // CH
US = 10
NO = N_OBS // NW
CG = 4000
NCG = NO // CG
UG = 10
UA = 8

_mesh = plsc.VectorSubcoreMesh(core_axis_name="c", subcore_axis_name="s")
_params = pltpu.CompilerParams(needs_layout_passes=False,
                               use_tc_tiling_on_sc=False)


@functools.partial(
    pl.kernel, mesh=_mesh, compiler_params=_params,
    out_type=jax.ShapeDtypeStruct((NW, NBINS), jnp.float32),
    scratch_types=[
        pltpu.VMEM((NBINS,), jnp.float32),
        pltpu.VMEM((CH,), jnp.int32),
        pltpu.VMEM((CH,), jnp.int32),
        pltpu.SemaphoreType.DMA,
        pltpu.SemaphoreType.DMA,
    ],
)
def _k1_histogram(idx_hbm, out_hbm, hist, idxa, idxb, sem0, sem1):
    c = lax.axis_index("c")
    s = lax.axis_index("s")
    wid = s * NC + c

    @plsc.parallel_loop(0, NBINS // L, 1, unroll=16)
    def _(i):
        hist[pl.ds(i * L, L)] = jnp.zeros((L,), jnp.float32)

    bufs = (idxa, idxb)
    sems = (sem0, sem1)
    tile_base = wid * NE
    cps = [None, None]
    cps[0] = pltpu.async_copy(idx_hbm.at[pl.ds(tile_base, CH)], idxa, sem0)
    for k in range(NCH):
        cur = k % 2
        if k + 1 < NCH:
            nxt = (k + 1) % 2
            cps[nxt] = pltpu.async_copy(
                idx_hbm.at[pl.ds(tile_base + (k + 1) * CH, CH)],
                bufs[nxt], sems[nxt])
        cps[cur].wait()
        buf = bufs[cur]

        @plsc.parallel_loop(0, CH // L, 1, unroll=US)
        def _(i):
            iv = buf[pl.ds(i * L, L)]
            plsc.addupdate_scatter(hist, [iv], jnp.ones((L,), jnp.float32))

    pltpu.sync_copy(hist, out_hbm.at[wid])


@functools.partial(
    pl.kernel, mesh=_mesh, compiler_params=_params,
    out_type=jax.ShapeDtypeStruct((N_OBS,), jnp.float32),
    scratch_types=[
        pltpu.VMEM((NBINS,), jnp.float32),
        pltpu.VMEM((CG,), jnp.int32),
        pltpu.VMEM((CG,), jnp.int32),
        pltpu.VMEM((SLICE,), jnp.float32),
        pltpu.VMEM((SLICE,), jnp.float32),
        pltpu.VMEM((NS * L,), jnp.float32),
        pltpu.VMEM_SHARED((NBINS,), jnp.float32),
        pltpu.VMEM_SHARED((NS * L,), jnp.float32),
        pltpu.SemaphoreType.DMA,
        pltpu.SemaphoreType.DMA,
        pltpu.SemaphoreType.DMA,
        pltpu.SemaphoreType.DMA,
    ],
)
def _k2_normalize_gather(parts_hbm, lam_hbm, wh_hbm, out_hbm,
                         ttab, wha, whb, tbuf, tbuf2, mbuf, st, smax,
                         sem0, sem1, sem2, sem3):
    c = lax.axis_index("c")
    s = lax.axis_index("s")
    wid = s * NC + c
    off = s * SLICE

    pltpu.sync_copy(parts_hbm.at[0, pl.ds(off, SLICE)], ttab.at[pl.ds(0, SLICE)])
    cp1 = pltpu.async_copy(parts_hbm.at[1, pl.ds(off, SLICE)], tbuf, sem0)
    cp2 = pltpu.async_copy(parts_hbm.at[2, pl.ds(off, SLICE)], tbuf2, sem1)
    for p in range(1, NW):
        use_a = (p % 2) == 1
        (cp1 if use_a else cp2).wait()
        src = tbuf if use_a else tbuf2

        @plsc.parallel_loop(0, SLICE // L, 1, unroll=UA)
        def _(j):
            q = j * L
            ttab[pl.ds(q, L)] = ttab[pl.ds(q, L)] + src[pl.ds(q, L)]

        if p + 2 < NW:
            if use_a:
                cp1 = pltpu.async_copy(parts_hbm.at[p + 2, pl.ds(off, SLICE)],
                                       tbuf, sem0)
            else:
                cp2 = pltpu.async_copy(parts_hbm.at[p + 2, pl.ds(off, SLICE)],
                                       tbuf2, sem1)

    @pl.when(s < NS - 1)
    def _():
        pltpu.sync_copy(lam_hbm.at[pl.ds(off, SLICE)],
                        ttab.at[pl.ds(2 * SLICE, SLICE)])

    @pl.when(s == NS - 1)
    def _():
        pltpu.sync_copy(lam_hbm.at[pl.ds(off, LAM_TAIL)],
                        ttab.at[pl.ds(2 * SLICE, LAM_TAIL)])

    def max_body(j, m):
        return jnp.maximum(m, ttab[pl.ds(j * L, L)])
    mv = plsc.parallel_loop(0, SLICE // L, 1, unroll=UA,
                            carry=jnp.zeros((L,), jnp.float32))(max_body)
    lmax = lax.reduce_max_p.bind(mv, axes=(0,))
    mbuf[pl.ds(0, L)] = jnp.full((L,), lmax)
    pltpu.sync_copy(mbuf.at[pl.ds(0, L)], smax.at[pl.ds(s * L, L)])
    plsc.subcore_barrier()
    pltpu.sync_copy(smax, mbuf)
    gv = mbuf[pl.ds(0, L)]
    for p in range(1, NS):
        gv = jnp.maximum(gv, mbuf[pl.ds(p * L, L)])
    inv = 1.0 / jnp.full((L,), lax.reduce_max_p.bind(gv, axes=(0,)))

    @plsc.parallel_loop(0, SLICE // L, 1, unroll=UA)
    def _(j):
        q = j * L
        d = ttab[pl.ds(q, L)]
        lm = ttab[pl.ds(2 * SLICE + q, L)]
        ttab[pl.ds(SLICE + q, L)] = (
            10.0 / (jnp.abs(lm * d * inv) + 1.0) + 1e-05)

    pltpu.sync_copy(ttab.at[pl.ds(SLICE, SLICE)], st.at[pl.ds(off, SLICE)])
    plsc.subcore_barrier()
    pltpu.sync_copy(st, ttab)

    whs = (wha, whb)
    outs = (tbuf, tbuf2)
    isems = (sem0, sem1)
    osems = (sem2, sem3)
    obase = wid * NO
    cin = [None, None]
    cout = [None, None]
    cin[0] = pltpu.async_copy(wh_hbm.at[pl.ds(obase, CG)], wha, sem0)
    for k in range(NCG):
        cur = k % 2
        if k + 1 < NCG:
            nxt = (k + 1) % 2
            cin[nxt] = pltpu.async_copy(
                wh_hbm.at[pl.ds(obase + (k + 1) * CG, CG)],
                whs[nxt], isems[nxt])
        cin[cur].wait()
        if k >= 2:
            cout[cur].wait()
        wh = whs[cur]
        ob = outs[cur]

        @plsc.parallel_loop(0, CG // L, 1, unroll=UG)
        def _(i):
            q = i * L
            iv = wh[pl.ds(q, L)]
            ob[pl.ds(q, L)] = plsc.load_gather(ttab, [iv])

        cout[cur] = pltpu.async_copy(
            ob.at[pl.ds(0, CG)], out_hbm.at[pl.ds(obase + k * CG, CG)],
            osems[cur])
    cout[0].wait()
    cout[1].wait()


def kernel(lam, idx, wh_o):
    idx_perm = jnp.reshape(
        jnp.transpose(jnp.reshape(idx, (2, NNZ // 128, 128)), (1, 0, 2)),
        (E,))
    parts = _k1_histogram(idx_perm)
    return _k2_normalize_gather(parts, lam, wh_o)
